# baseline (speedup 1.0000x reference)
import functools

import jax
import jax.numpy as jnp
from jax import lax
from jax.experimental import pallas as pl
from jax.experimental.pallas import tpu as pltpu
from jax.experimental.pallas import tpu_sc as plsc

B = 16384
D = 64
NC = 2
NS = 16
NW = NC * NS
BPW = B // NW
GROUPS = BPW // 16


def _neg_sqrt(s):
    xs = jnp.maximum(s, jnp.float32(1e-37))
    i = lax.bitcast_convert_type(xs, jnp.int32)
    r = lax.bitcast_convert_type(jnp.int32(0x5F3759DF) - (i >> 1), jnp.float32)
    for _ in range(3):
        r = r * (jnp.float32(1.5) - jnp.float32(0.5) * xs * r * r)
    return -(xs * r)


def _body(x_hbm, y_hbm, tab_hbm, out_hbm,
          idx_x, idx_y, rows_x, rows_y, out_v, sem_x, sem_y):
    c = lax.axis_index("c")
    s = lax.axis_index("s")
    wid = s * NC + c
    base = wid * BPW

    pltpu.sync_copy(x_hbm.at[pl.ds(base, BPW)], idx_x)
    pltpu.sync_copy(y_hbm.at[pl.ds(base, BPW)], idx_y)
    cp_x = pltpu.async_copy(tab_hbm.at[idx_x], rows_x, sem_x)
    cp_y = pltpu.async_copy(tab_hbm.at[idx_y], rows_y, sem_y)
    cp_x.wait()
    cp_y.wait()

    lane = lax.iota(jnp.int32, 16)

    def g_body(g, carry):
        row = g * 16 + lane
        acc = jnp.zeros((16,), jnp.float32)
        for dd in range(D):
            col = jnp.full((16,), dd, jnp.int32)
            xv = plsc.load_gather(rows_x, [row, col])
            yv = plsc.load_gather(rows_y, [row, col])
            t = xv - yv
            acc = acc + t * t
        out_v[pl.ds(g * 16, 16)] = _neg_sqrt(acc)
        return carry

    lax.fori_loop(0, GROUPS, g_body, 0)
    pltpu.sync_copy(out_v, out_hbm.at[pl.ds(base, BPW)])


@jax.jit
def kernel(x, y, entity_embeddings):
    mesh = plsc.VectorSubcoreMesh(core_axis_name="c", subcore_axis_name="s")
    run = functools.partial(
        pl.kernel,
        out_type=jax.ShapeDtypeStruct((B,), jnp.float32),
        mesh=mesh,
        compiler_params=pltpu.CompilerParams(
            use_tc_tiling_on_sc=False, needs_layout_passes=False),
        scratch_types=[
            pltpu.VMEM((BPW,), jnp.int32),
            pltpu.VMEM((BPW,), jnp.int32),
            pltpu.VMEM((BPW, D), jnp.float32),
            pltpu.VMEM((BPW, D), jnp.float32),
            pltpu.VMEM((BPW,), jnp.float32),
            pltpu.SemaphoreType.DMA,
            pltpu.SemaphoreType.DMA,
        ],
    )(_body)
    return run(x.astype(jnp.int32), y.astype(jnp.int32), entity_embeddings)


# speedup vs baseline: 1.4738x; 1.4738x over previous
import functools

import jax
import jax.numpy as jnp
from jax import lax
from jax.experimental import pallas as pl
from jax.experimental.pallas import tpu as pltpu
from jax.experimental.pallas import tpu_sc as plsc

B = 16384
D = 64
NC = 2
NS = 16
NW = NC * NS
BPW = B // NW
C = 32
NCHUNK = BPW // C


def _neg_sqrt(s):
    xs = jnp.maximum(s, jnp.float32(1e-37))
    i = lax.bitcast_convert_type(xs, jnp.int32)
    r = lax.bitcast_convert_type(jnp.int32(0x5F3759DF) - (i >> 1), jnp.float32)
    for _ in range(3):
        r = r * (jnp.float32(1.5) - jnp.float32(0.5) * xs * r * r)
    return -(xs * r)


def _body(x_hbm, y_hbm, tab_hbm, out_hbm,
          idx_xv, idx_yv, stage_x, stage_y, out_v,
          sem_x, sem_y):
    c = lax.axis_index("c")
    s = lax.axis_index("s")
    wid = s * NC + c
    base = wid * BPW

    pltpu.sync_copy(x_hbm.at[pl.ds(base, BPW)], idx_xv)
    pltpu.sync_copy(y_hbm.at[pl.ds(base, BPW)], idx_yv)

    lane = lax.iota(jnp.int32, 16)

    def chunk_body(ch, carry):
        vecs = []
        for g2 in range(C // 16):
            xvec = idx_xv[pl.ds(ch * C + g2 * 16, 16)]
            yvec = idx_yv[pl.ds(ch * C + g2 * 16, 16)]
            vecs.append((xvec, yvec))
            for i in range(16):
                rx0 = pl.multiple_of((xvec[i] >> 3) << 3, 8)
                ry0 = pl.multiple_of((yvec[i] >> 3) << 3, 8)
                pltpu.async_copy(tab_hbm.at[pl.ds(rx0, 8)],
                                 stage_x.at[g2 * 16 + i], sem_x)
                pltpu.async_copy(tab_hbm.at[pl.ds(ry0, 8)],
                                 stage_y.at[g2 * 16 + i], sem_y)
        for i in range(C):
            pltpu.make_async_copy(tab_hbm.at[pl.ds(0, 8)],
                                  stage_x.at[i], sem_x).wait()
            pltpu.make_async_copy(tab_hbm.at[pl.ds(0, 8)],
                                  stage_y.at[i], sem_y).wait()

        for g2 in range(C // 16):
            slot = g2 * 16 + lane
            ivx, ivy = vecs[g2]
            r8x = lax.bitwise_and(ivx, jnp.int32(7))
            r8y = lax.bitwise_and(ivy, jnp.int32(7))
            acc = jnp.zeros((16,), jnp.float32)
            for dd in range(D):
                col = jnp.full((16,), dd, jnp.int32)
                xv = plsc.load_gather(stage_x, [slot, r8x, col])
                yv = plsc.load_gather(stage_y, [slot, r8y, col])
                t = xv - yv
                acc = acc + t * t
            out_v[pl.ds(ch * C + g2 * 16, 16)] = _neg_sqrt(acc)
        return carry

    lax.fori_loop(0, NCHUNK, chunk_body, 0)
    pltpu.sync_copy(out_v, out_hbm.at[pl.ds(base, BPW)])


@jax.jit
def kernel(x, y, entity_embeddings):
    mesh = plsc.VectorSubcoreMesh(core_axis_name="c", subcore_axis_name="s")
    run = functools.partial(
        pl.kernel,
        out_type=jax.ShapeDtypeStruct((B,), jnp.float32),
        mesh=mesh,
        compiler_params=pltpu.CompilerParams(
            use_tc_tiling_on_sc=True, needs_layout_passes=False),
        scratch_types=[
            pltpu.VMEM((BPW,), jnp.int32),
            pltpu.VMEM((BPW,), jnp.int32),
            pltpu.VMEM((C, 8, D), jnp.float32),
            pltpu.VMEM((C, 8, D), jnp.float32),
            pltpu.VMEM((BPW,), jnp.float32),
            pltpu.SemaphoreType.DMA,
            pltpu.SemaphoreType.DMA,
        ],
    )(_body)
    return run(x.astype(jnp.int32), y.astype(jnp.int32), entity_embeddings)
